# Initial kernel scaffold; baseline (speedup 1.0000x reference)
import jax
import jax.numpy as jnp
from jax.experimental import pallas as pl


def kernel(x, gamma, W_r, Wg, Wu, Wd):
    raise NotImplementedError("write your pallas kernel here")


# speedup vs baseline: 1.4432x; 1.4432x over previous
import jax
import jax.numpy as jnp
from jax.experimental import pallas as pl
from jax.experimental.pallas import tpu as pltpu

T = 2048
D = 1024
E = 8
FF = 1024
EPS = 1e-5


def _stage1_body(x_ref, gamma_ref, wr_ref, xnb_ref, rw_ref):
    x = x_ref[...]
    var = jnp.mean(x * x, axis=-1, keepdims=True)
    xn = x * jax.lax.rsqrt(var + EPS) * gamma_ref[...][None, :]
    logits = jnp.dot(xn, wr_ref[...], preferred_element_type=jnp.float32)
    scores = jax.nn.sigmoid(logits)
    i1 = jnp.argmax(scores, axis=-1)
    v1 = jnp.max(scores, axis=-1)
    cols = jax.lax.broadcasted_iota(jnp.int32, scores.shape, 1)
    masked = jnp.where(cols == i1[:, None], -jnp.inf, scores)
    i2 = jnp.argmax(masked, axis=-1)
    v2 = jnp.max(masked, axis=-1)
    denom = v1 + v2 + 1e-20
    w1 = (v1 / denom)[:, None]
    w2 = (v2 / denom)[:, None]
    rw = jnp.where(cols == i1[:, None], w1, 0.0) + jnp.where(
        cols == i2[:, None], w2, 0.0)
    rw_ref[...] = rw
    xnb_ref[...] = xn.astype(jnp.bfloat16)


def _stage2_body(x_ref, xnb_ref, rw_ref, wg_ref, wu_ref, wd_ref, out_ref):
    e = pl.program_id(0)

    @pl.when(e == 0)
    def _():
        out_ref[...] = x_ref[...]

    rw = rw_ref[...]
    cols = jax.lax.broadcasted_iota(jnp.int32, rw.shape, 1)
    w = jnp.sum(jnp.where(cols == e, rw, 0.0), axis=-1, keepdims=True)
    xb = xnb_ref[...]
    g = jnp.dot(xb, wg_ref[0], preferred_element_type=jnp.float32)
    u = jnp.dot(xb, wu_ref[0], preferred_element_type=jnp.float32)
    h = (g * jax.nn.sigmoid(g) * u).astype(jnp.bfloat16)
    y = jnp.dot(h, wd_ref[0], preferred_element_type=jnp.float32)
    out_ref[...] += w * y


def kernel(x, gamma, W_r, Wg, Wu, Wd):
    xnb, rw = pl.pallas_call(
        _stage1_body,
        grid=(8,),
        in_specs=[
            pl.BlockSpec((T // 8, D), lambda i: (i, 0)),
            pl.BlockSpec((D,), lambda i: (0,)),
            pl.BlockSpec((D, E), lambda i: (0, 0)),
        ],
        out_specs=[
            pl.BlockSpec((T // 8, D), lambda i: (i, 0)),
            pl.BlockSpec((T // 8, E), lambda i: (i, 0)),
        ],
        out_shape=[
            jax.ShapeDtypeStruct((T, D), jnp.bfloat16),
            jax.ShapeDtypeStruct((T, E), jnp.float32),
        ],
    )(x, gamma, W_r)

    wg_b = Wg.astype(jnp.bfloat16)
    wu_b = Wu.astype(jnp.bfloat16)
    wd_b = Wd.astype(jnp.bfloat16)

    out = pl.pallas_call(
        _stage2_body,
        grid=(E,),
        in_specs=[
            pl.BlockSpec((T, D), lambda e: (0, 0)),
            pl.BlockSpec((T, D), lambda e: (0, 0)),
            pl.BlockSpec((T, E), lambda e: (0, 0)),
            pl.BlockSpec((1, D, FF), lambda e: (e, 0, 0)),
            pl.BlockSpec((1, D, FF), lambda e: (e, 0, 0)),
            pl.BlockSpec((1, FF, D), lambda e: (e, 0, 0)),
        ],
        out_specs=pl.BlockSpec((T, D), lambda e: (0, 0)),
        out_shape=jax.ShapeDtypeStruct((T, D), jnp.float32),
    )(x, xnb, rw, wg_b, wu_b, wd_b)
    return out


# speedup vs baseline: 1.6139x; 1.1183x over previous
import functools

import jax
import jax.numpy as jnp
from jax import lax
from jax.experimental import pallas as pl
from jax.experimental.pallas import tpu as pltpu
from jax.experimental.pallas import tpu_sc as plsc

T = 2048
D = 1024
E = 8
FF = 1024
EPS = 1e-5

R = 256
NBLK = 24
NPAD = NBLK * R
NW = 32
TPW = T // NW


def _cumsum_lanes(m):
    n = m.shape[-1]
    s = 1
    while s < n:
        shifted = jnp.pad(m, ((0, 0), (s, 0)))[:, :n]
        m = m + shifted
        s *= 2
    return m


def _stage1_body(x_ref, gamma_ref, wr_ref, xnb_ref, d0_ref, d1_ref,
                 w0_ref, w1_ref, bexp_ref, bval_ref):
    x = x_ref[...]
    var = jnp.mean(x * x, axis=-1, keepdims=True)
    xn = x * lax.rsqrt(var + EPS) * gamma_ref[...][None, :]
    xnb_ref[...] = xn

    logits = jnp.dot(xn, wr_ref[...], preferred_element_type=jnp.float32)
    scores = jax.nn.sigmoid(logits)
    i1 = jnp.argmax(scores, axis=-1)
    v1 = jnp.max(scores, axis=-1, keepdims=True)
    cols = lax.broadcasted_iota(jnp.int32, scores.shape, 1)
    masked = jnp.where(cols == i1[:, None], -jnp.inf, scores)
    i2 = jnp.argmax(masked, axis=-1)
    v2 = jnp.max(masked, axis=-1, keepdims=True)
    den = v1 + v2 + 1e-20
    w0_ref[...] = v1 / den
    w1_ref[...] = v2 / den

    erow = lax.broadcasted_iota(jnp.int32, (E, T), 0)
    oh1 = (erow == i1[None, :]).astype(jnp.float32)
    oh2 = (erow == i2[None, :]).astype(jnp.float32)
    m = oh1 + oh2
    cum = _cumsum_lanes(m)
    excl = cum - m
    counts = cum[:, T - 1][None, :]
    nblk_e = (counts.astype(jnp.int32) + (R - 1)) // R
    ends = _cumsum_lanes(nblk_e)
    baseblk = ends - nblk_e
    base_rows = (baseblk * R).astype(jnp.float32)
    base_col = base_rows.reshape(E, 1)
    d0 = jnp.sum(oh1 * (base_col + excl), axis=0)
    d1 = jnp.sum(oh2 * (base_col + excl), axis=0)
    d0_ref[...] = d0.astype(jnp.int32)
    d1_ref[...] = d1.astype(jnp.int32)

    brow = lax.broadcasted_iota(jnp.int32, (NBLK, E), 0)
    ends_b = jnp.broadcast_to(ends, (NBLK, E))
    bexp = jnp.sum((brow >= ends_b).astype(jnp.int32), axis=1)
    bval_ref[...] = (bexp < E).astype(jnp.int32)
    bexp_ref[...] = jnp.minimum(bexp, E - 1)


def _stage1(x, gamma, W_r):
    return pl.pallas_call(
        _stage1_body,
        grid=(1,),
        in_specs=[
            pl.BlockSpec((T, D), lambda i: (0, 0)),
            pl.BlockSpec((D,), lambda i: (0,)),
            pl.BlockSpec((D, E), lambda i: (0, 0)),
        ],
        out_specs=[
            pl.BlockSpec((T, D), lambda i: (0, 0)),
            pl.BlockSpec((T,), lambda i: (0,)),
            pl.BlockSpec((T,), lambda i: (0,)),
            pl.BlockSpec((T, 1), lambda i: (0, 0)),
            pl.BlockSpec((T, 1), lambda i: (0, 0)),
            pl.BlockSpec((NBLK,), lambda i: (0,)),
            pl.BlockSpec((NBLK,), lambda i: (0,)),
        ],
        out_shape=[
            jax.ShapeDtypeStruct((T, D), jnp.float32),
            jax.ShapeDtypeStruct((T,), jnp.int32),
            jax.ShapeDtypeStruct((T,), jnp.int32),
            jax.ShapeDtypeStruct((T, 1), jnp.float32),
            jax.ShapeDtypeStruct((T, 1), jnp.float32),
            jax.ShapeDtypeStruct((NBLK,), jnp.int32),
            jax.ShapeDtypeStruct((NBLK,), jnp.int32),
        ],
    )(x, gamma, W_r)


def _sc_scatter_body(xnb_hbm, d0_hbm, d1_hbm, xs_hbm, i0_v, i1_v, rows_v, sem):
    wid = lax.axis_index("s") * 2 + lax.axis_index("c")
    base = wid * TPW
    pltpu.sync_copy(d0_hbm.at[pl.ds(base, TPW)], i0_v)
    pltpu.sync_copy(d1_hbm.at[pl.ds(base, TPW)], i1_v)
    pltpu.sync_copy(xnb_hbm.at[pl.ds(base, TPW)], rows_v)
    pltpu.async_copy(rows_v, xs_hbm.at[i0_v], sem).wait()
    pltpu.async_copy(rows_v, xs_hbm.at[i1_v], sem).wait()


def _sc_gather_body(ys_hbm, d0_hbm, d1_hbm, y0_hbm, y1_hbm,
                    i0_v, i1_v, r0_v, r1_v, sem):
    wid = lax.axis_index("s") * 2 + lax.axis_index("c")
    half = TPW // 2
    for h in range(2):
        base = wid * TPW + h * half
        pltpu.sync_copy(d0_hbm.at[pl.ds(base, half)], i0_v)
        pltpu.sync_copy(d1_hbm.at[pl.ds(base, half)], i1_v)
        cp0 = pltpu.async_copy(ys_hbm.at[i0_v], r0_v, sem)
        cp1 = pltpu.async_copy(ys_hbm.at[i1_v], r1_v, sem)
        cp0.wait()
        cp1.wait()
        pltpu.sync_copy(r0_v, y0_hbm.at[pl.ds(base, half)])
        pltpu.sync_copy(r1_v, y1_hbm.at[pl.ds(base, half)])


@functools.lru_cache(maxsize=None)
def _sc_kernels():
    mesh = plsc.VectorSubcoreMesh(core_axis_name="c", subcore_axis_name="s")
    scatter = pl.kernel(
        _sc_scatter_body,
        out_type=jax.ShapeDtypeStruct((NPAD, D), jnp.float32),
        mesh=mesh,
        scratch_types=[
            pltpu.VMEM((TPW,), jnp.int32),
            pltpu.VMEM((TPW,), jnp.int32),
            pltpu.VMEM((TPW, D), jnp.float32),
            pltpu.SemaphoreType.DMA,
        ],
    )
    gather = pl.kernel(
        _sc_gather_body,
        out_type=[
            jax.ShapeDtypeStruct((T, D), jnp.float32),
            jax.ShapeDtypeStruct((T, D), jnp.float32),
        ],
        mesh=mesh,
        scratch_types=[
            pltpu.VMEM((TPW // 2,), jnp.int32),
            pltpu.VMEM((TPW // 2,), jnp.int32),
            pltpu.VMEM((TPW // 2, D), jnp.float32),
            pltpu.VMEM((TPW // 2, D), jnp.float32),
            pltpu.SemaphoreType.DMA,
        ],
    )
    return scatter, gather


def _stage2_body(bexp_ref, bval_ref, xs_ref, wg_ref, wu_ref, wd_ref, ys_ref):
    b = pl.program_id(0)

    @pl.when(bval_ref[b] == 1)
    def _():
        xb = xs_ref[...].astype(jnp.bfloat16)
        g = jnp.dot(xb, wg_ref[0], preferred_element_type=jnp.float32)
        u = jnp.dot(xb, wu_ref[0], preferred_element_type=jnp.float32)
        h = (g * jax.nn.sigmoid(g) * u).astype(jnp.bfloat16)
        y = jnp.dot(h, wd_ref[0], preferred_element_type=jnp.float32)
        ys_ref[...] = y


def _stage2(bexp, bval, xs, wg_b, wu_b, wd_b):
    grid_spec = pltpu.PrefetchScalarGridSpec(
        num_scalar_prefetch=2,
        grid=(NBLK,),
        in_specs=[
            pl.BlockSpec((R, D),
                         lambda b, be, bv: (jnp.where(bv[b] == 1, b, 0), 0)),
            pl.BlockSpec((1, D, FF), lambda b, be, bv: (be[b], 0, 0)),
            pl.BlockSpec((1, D, FF), lambda b, be, bv: (be[b], 0, 0)),
            pl.BlockSpec((1, FF, D), lambda b, be, bv: (be[b], 0, 0)),
        ],
        out_specs=pl.BlockSpec((R, D), lambda b, be, bv: (b, 0)),
    )
    return pl.pallas_call(
        _stage2_body,
        grid_spec=grid_spec,
        out_shape=jax.ShapeDtypeStruct((NPAD, D), jnp.float32),
    )(bexp, bval, xs, wg_b, wu_b, wd_b)


def _combine_body(x_ref, y0_ref, y1_ref, w0_ref, w1_ref, out_ref):
    out_ref[...] = (x_ref[...] + w0_ref[...] * y0_ref[...]
                    + w1_ref[...] * y1_ref[...])


def _combine(x, y0, y1, w0, w1):
    return pl.pallas_call(
        _combine_body,
        grid=(1,),
        in_specs=[
            pl.BlockSpec((T, D), lambda i: (0, 0)),
            pl.BlockSpec((T, D), lambda i: (0, 0)),
            pl.BlockSpec((T, D), lambda i: (0, 0)),
            pl.BlockSpec((T, 1), lambda i: (0, 0)),
            pl.BlockSpec((T, 1), lambda i: (0, 0)),
        ],
        out_specs=pl.BlockSpec((T, D), lambda i: (0, 0)),
        out_shape=jax.ShapeDtypeStruct((T, D), jnp.float32),
    )(x, y0, y1, w0, w1)


def kernel(x, gamma, W_r, Wg, Wu, Wd):
    xnb, d0, d1, w0, w1, bexp, bval = _stage1(x, gamma, W_r)
    _sc_scatter, _sc_gather = _sc_kernels()
    xs = _sc_scatter(xnb, d0, d1)
    wg_b = Wg.astype(jnp.bfloat16)
    wu_b = Wu.astype(jnp.bfloat16)
    wd_b = Wd.astype(jnp.bfloat16)
    ys = _stage2(bexp, bval, xs, wg_b, wu_b, wd_b)
    y0, y1 = _sc_gather(ys, d0, d1)
    return _combine(x, y0, y1, w0, w1)
